# baseline (speedup 1.0000x reference)
import jax
import jax.numpy as jnp
from jax.experimental import pallas as pl

_REC_N = 512
_LIG_N = 64


def _segment_softmax(logits, seg, num):
    mx = jax.ops.segment_max(logits, seg, num_segments=num)
    mx = jnp.where(jnp.isfinite(mx), mx, 0.0)
    e = jnp.exp(logits - mx[seg])
    den = jax.ops.segment_sum(e, seg, num_segments=num)
    return e / (den[seg] + 1e-9)


def _mp_layer(h, x, src, dst, ew, Wm, Wa, Wh, N):
    d = x[dst] - x[src]
    r = jnp.sqrt(jnp.sum(d * d, axis=-1, keepdims=True) + 1e-12)
    feat = jnp.concatenate([h[src], h[dst], ew, r], axis=-1)
    m = jax.nn.relu(feat @ Wm)
    a = _segment_softmax((m @ Wa)[:, 0], dst, N)
    agg = jax.ops.segment_sum(a[:, None] * m, dst, num_segments=N)
    return jax.nn.relu(jnp.concatenate([h, agg], axis=-1) @ Wh)


def _identity_pallas(x):
    def body(x_ref, o_ref):
        o_ref[...] = x_ref[...]
    return pl.pallas_call(
        body, out_shape=jax.ShapeDtypeStruct(x.shape, x.dtype))(x)


def kernel(rec_x, rec_f, rec_vec, rec_edge_index, rec_edge_w, lig_x, lig_f, lig_edge_index, lig_edge_w, rWm0, rWa0, rWh0, rWm1, rWa1, rWh1, rWo, lWm0, lWa0, lWh0, lWm1, lWa1, lWh1, lWo, cWm0, cWa0, cWh0, cWm1, cWa1, cWh1, cWmf, cWdf):
    rs, rd = rec_edge_index[0], rec_edge_index[1]
    ls, ld = lig_edge_index[0], lig_edge_index[1]
    vnorm = jnp.sqrt(jnp.sum(rec_vec[:, 0, :] ** 2, axis=-1, keepdims=True) + 1e-12)
    h = jnp.concatenate([rec_f, vnorm], axis=-1)
    h = _mp_layer(h, rec_x, rs, rd, rec_edge_w, rWm0, rWa0, rWh0, _REC_N)
    h = _mp_layer(h, rec_x, rs, rd, rec_edge_w, rWm1, rWa1, rWh1, _REC_N)
    h_rec = h @ rWo
    h = lig_f
    h = _mp_layer(h, lig_x, ls, ld, lig_edge_w, lWm0, lWa0, lWh0, _LIG_N)
    h = _mp_layer(h, lig_x, ls, ld, lig_edge_w, lWm1, lWa1, lWh1, _LIG_N)
    h_lig = h @ lWo
    N = _REC_N + _LIG_N
    ii = jnp.repeat(jnp.arange(_REC_N, dtype=rs.dtype), _LIG_N)
    jj = jnp.tile(jnp.arange(_LIG_N, dtype=rs.dtype) + _REC_N, _REC_N)
    src = jnp.concatenate([rs, ls + _REC_N, ii, jj])
    dst = jnp.concatenate([rd, ld + _REC_N, jj, ii])
    Er, El, Ec = rs.shape[0], ls.shape[0], ii.shape[0]
    et = jnp.concatenate([
        jnp.tile(jnp.array([[1.0, 0.0, 0.0]], jnp.float32), (Er, 1)),
        jnp.tile(jnp.array([[0.0, 1.0, 0.0]], jnp.float32), (El, 1)),
        jnp.tile(jnp.array([[0.0, 0.0, 1.0]], jnp.float32), (2 * Ec, 1))], axis=0)
    x = jnp.concatenate([rec_x, lig_x], axis=0)
    hc = jnp.concatenate([h_rec, h_lig], axis=0)
    hc = _mp_layer(hc, x, src, dst, et, cWm0, cWa0, cWh0, N)
    hc = _mp_layer(hc, x, src, dst, et, cWm1, cWa1, cWh1, N)
    d = x[dst] - x[src]
    r = jnp.sqrt(jnp.sum(d * d, axis=-1, keepdims=True) + 1e-12)
    feat = jnp.concatenate([hc[src], hc[dst], et, r], axis=-1)
    mf = jax.nn.relu(feat @ cWmf)
    score = mf @ cWdf
    unit = d / (r + 1e-9)
    upd = jax.ops.segment_sum(score * unit, dst, num_segments=N)
    x_new = x + upd
    lig_new = _identity_pallas(x_new[_REC_N:])
    return jnp.stack([lig_x, lig_new])[None, :]


# speedup vs baseline: 3.6470x; 3.6470x over previous
import jax, jax.numpy as jnp
from jax.experimental import pallas as pl


def _idp(x):
    def body(x_ref, o_ref):
        o_ref[...] = x_ref[...]
    return pl.pallas_call(body, out_shape=jax.ShapeDtypeStruct(x.shape, x.dtype))(x)

REC_N, LIG_N = 512, 64


def mp_layer_sparse(h, src, dst, r, ew, Wm, Wa, Wh, N, F):
    Hs = h @ Wm[0:F]
    Hd = h @ Wm[F:2 * F]
    c = ew @ Wm[2 * F:2 * F + ew.shape[1]] + r[:, None] * Wm[2 * F + ew.shape[1]]
    m = jax.nn.relu(Hs[src] + Hd[dst] + c)
    w = jnp.exp(m @ Wa[:, 0])
    den0 = jax.ops.segment_sum(w, dst, num_segments=N)
    a = w / den0[dst]
    agg = jax.ops.segment_sum(a[:, None] * m, dst, num_segments=N)
    return jax.nn.relu(jnp.concatenate([h, agg], axis=-1) @ Wh)


def mp_layer_combined(h, src_sp, dst_sp, r_sp, ne_rec, ne_lig, rr, Wm, Wa, Wh):
    N, F = REC_N + LIG_N, 128
    Hs = h @ Wm[0:F]
    Hd = h @ Wm[F:2 * F]
    Wet = Wm[2 * F:2 * F + 3]
    wr = Wm[2 * F + 3]
    c_sp = jnp.concatenate([
        jnp.broadcast_to(Wet[0], (ne_rec, F)),
        jnp.broadcast_to(Wet[1], (ne_lig, F))], axis=0) + r_sp[:, None] * wr
    m_sp = jax.nn.relu(Hs[src_sp] + Hd[dst_sp] + c_sp)
    w_sp = jnp.exp(m_sp @ Wa[:, 0])
    den_sp = jax.ops.segment_sum(w_sp, dst_sp, num_segments=N)
    base = Wet[2] + rr[:, :, None] * wr
    m1 = jax.nn.relu(Hs[:REC_N, None, :] + Hd[None, REC_N:, :] + base)
    w1 = jnp.exp(m1 @ Wa[:, 0])
    m2 = jax.nn.relu(Hs[None, REC_N:, :] + Hd[:REC_N, None, :] + base)
    w2 = jnp.exp(m2 @ Wa[:, 0])
    den = den_sp.at[REC_N:].add(w1.sum(axis=0)).at[:REC_N].add(w2.sum(axis=1))
    a_sp = w_sp / den[dst_sp]
    agg = jax.ops.segment_sum(a_sp[:, None] * m_sp, dst_sp, num_segments=N)
    a1 = w1 / den[None, REC_N:]
    agg = agg.at[REC_N:].add(jnp.einsum('ij,ijf->jf', a1, m1))
    a2 = w2 / den[:REC_N, None]
    agg = agg.at[:REC_N].add(jnp.einsum('ij,ijf->if', a2, m2))
    return jax.nn.relu(jnp.concatenate([h, agg], axis=-1) @ Wh)


def final_conv(h, src_sp, dst_sp, r_sp, d_sp, rr, dd, x, Wm, Wd):
    N, F = REC_N + LIG_N, 128
    Hs = h @ Wm[0:F]
    Hd = h @ Wm[F:2 * F]
    Wet = Wm[2 * F:2 * F + 3]
    wr = Wm[2 * F + 3]
    c_sp = jnp.broadcast_to(Wet[1], (src_sp.shape[0], F)) + r_sp[:, None] * wr
    mf = jax.nn.relu(Hs[src_sp] + Hd[dst_sp] + c_sp)
    score = mf @ Wd[:, 0]
    unit = d_sp / (r_sp[:, None] + 1e-9)
    upd_lig = jax.ops.segment_sum(score[:, None] * unit, dst_sp - REC_N,
                                  num_segments=LIG_N)
    base = Wet[2] + rr[:, :, None] * wr
    mf1 = jax.nn.relu(Hs[:REC_N, None, :] + Hd[None, REC_N:, :] + base)
    sc1 = mf1 @ Wd[:, 0]
    unit1 = dd / (rr[:, :, None] + 1e-9)
    upd_lig = upd_lig + jnp.einsum('ij,ijc->jc', sc1, unit1)
    return x[REC_N:] + upd_lig


def kernel(rec_x, rec_f, rec_vec, rec_edge_index, rec_edge_w, lig_x, lig_f,
        lig_edge_index, lig_edge_w, rWm0, rWa0, rWh0, rWm1, rWa1, rWh1, rWo,
        lWm0, lWa0, lWh0, lWm1, lWa1, lWh1, lWo, cWm0, cWa0, cWh0, cWm1,
        cWa1, cWh1, cWmf, cWdf):
    rs, rd = rec_edge_index[0], rec_edge_index[1]
    ls, ld = lig_edge_index[0], lig_edge_index[1]
    dr = rec_x[rd] - rec_x[rs]
    r_rec = jnp.sqrt(jnp.sum(dr * dr, -1) + 1e-12)
    dl = lig_x[ld] - lig_x[ls]
    r_lig = jnp.sqrt(jnp.sum(dl * dl, -1) + 1e-12)
    dd = lig_x[None, :, :] - rec_x[:, None, :]
    rr = jnp.sqrt(jnp.sum(dd * dd, -1) + 1e-12)

    vnorm = jnp.sqrt(jnp.sum(rec_vec[:, 0, :] ** 2, -1, keepdims=True) + 1e-12)
    h = jnp.concatenate([rec_f, vnorm], axis=-1)
    h = mp_layer_sparse(h, rs, rd, r_rec, rec_edge_w, rWm0, rWa0, rWh0, REC_N, 129)
    h = mp_layer_sparse(h, rs, rd, r_rec, rec_edge_w, rWm1, rWa1, rWh1, REC_N, 128)
    h_rec = h @ rWo
    h = lig_f
    h = mp_layer_sparse(h, ls, ld, r_lig, lig_edge_w, lWm0, lWa0, lWh0, LIG_N, 64)
    h = mp_layer_sparse(h, ls, ld, r_lig, lig_edge_w, lWm1, lWa1, lWh1, LIG_N, 64)
    h_lig = h @ lWo

    hc = jnp.concatenate([h_rec, h_lig], axis=0)
    src_sp = jnp.concatenate([rs, ls + REC_N])
    dst_sp = jnp.concatenate([rd, ld + REC_N])
    r_sp = jnp.concatenate([r_rec, r_lig])
    Er, El = rs.shape[0], ls.shape[0]
    hc = mp_layer_combined(hc, src_sp, dst_sp, r_sp, Er, El, rr, cWm0, cWa0, cWh0)
    hc = mp_layer_combined(hc, src_sp, dst_sp, r_sp, Er, El, rr, cWm1, cWa1, cWh1)

    x = jnp.concatenate([rec_x, lig_x], axis=0)
    lig_new = final_conv(hc, ls + REC_N, ld + REC_N, r_lig, dl, rr, dd,
                         x, cWmf, cWdf)
    return jnp.stack([lig_x, _idp(lig_new)])[None, :]


# speedup vs baseline: 10.8201x; 2.9669x over previous
import functools

import jax
import jax.numpy as jnp
from jax import lax
from jax.experimental import pallas as pl
from jax.experimental.pallas import tpu as pltpu
from jax.experimental.pallas import tpu_sc as plsc

REC_N, LIG_N = 512, 64
_NC, _NS, _L = 2, 16, 16
_FH = 64


def _zero_vmem(ref, n16):
    zv = jnp.zeros((_L,), jnp.float32)

    def body(i, carry):
        ref[pl.ds(i * _L, _L)] = zv
        return carry
    lax.fori_loop(0, n16, body, 0)


@functools.cache
def _sc_edge_pass(E, N, K, FO, final):
    Ec = E // 16
    nb = Ec // _L
    FH = FO // 2
    assert Ec % _L == 0

    mesh = plsc.VectorSubcoreMesh(core_axis_name="c", subcore_axis_name="s",
                                  num_cores=_NC, num_subcores=_NS)
    if final:
        out_type = jax.ShapeDtypeStruct((16, 192), jnp.float32)
    else:
        out_type = (jax.ShapeDtypeStruct((32, FH * N), jnp.float32),
                    jax.ShapeDtypeStruct((16, N), jnp.float32))

    scratch = [
        pltpu.VMEM((FH * N,), jnp.float32),
        pltpu.VMEM((FH * N,), jnp.float32),
        pltpu.VMEM((Ec,), jnp.int32),
        pltpu.VMEM((Ec,), jnp.int32),
        pltpu.VMEM((K * Ec,), jnp.float32),
        pltpu.VMEM((K * FH * _L,), jnp.float32),
        pltpu.VMEM((FH * _L,), jnp.float32),
        pltpu.VMEM((Ec,), jnp.float32),
        pltpu.VMEM((Ec,), jnp.float32),
        pltpu.VMEM((Ec,), jnp.float32),
        pltpu.VMEM_SHARED((_NS, Ec), jnp.float32),
    ]
    if final:
        scratch.append(pltpu.VMEM((3 * Ec,), jnp.float32))
        scratch.append(pltpu.VMEM((192,), jnp.float32))
    else:
        scratch.append(pltpu.VMEM((FH * N,), jnp.float32))
        scratch.append(pltpu.VMEM((N,), jnp.float32))

    def body(*refs):
        if final:
            (hsT, hdT, srcg, dstg, gT, wg, wa, unitT, out,
             hs_v, hd_v, idxs, idxd, g_v, wg_v, wa_v, l_v, lp_v, w_v, sh,
             unit_v, upd_v) = refs
        else:
            (hsT, hdT, srcg, dstg, gT, wg, wa, u_out, den_out,
             hs_v, hd_v, idxs, idxd, g_v, wg_v, wa_v, l_v, lp_v, w_v, sh,
             u_v, den_v) = refs

        c = lax.axis_index("c")
        s = lax.axis_index("s")
        fhalf = lax.rem(s, 2)
        echunk = lax.div(s, 2)
        chunk = c * 8 + echunk
        e0 = chunk * Ec

        pltpu.sync_copy(hsT.at[pl.ds(fhalf * FH * N, FH * N)], hs_v)
        pltpu.sync_copy(hdT.at[pl.ds(fhalf * FH * N, FH * N)], hd_v)
        pltpu.sync_copy(srcg.at[pl.ds(e0, Ec)], idxs)
        pltpu.sync_copy(dstg.at[pl.ds(e0, Ec)], idxd)
        for k in range(K):
            pltpu.sync_copy(gT.at[pl.ds(k * E + e0, Ec)],
                            g_v.at[pl.ds(k * Ec, Ec)])
        pltpu.sync_copy(wg.at[pl.ds(fhalf * K * FH * _L, K * FH * _L)],
                        wg_v)
        pltpu.sync_copy(wa.at[pl.ds(fhalf * FH * _L, FH * _L)], wa_v)
        if final:
            for k in range(3):
                pltpu.sync_copy(unitT.at[pl.ds(k * E + e0, Ec)],
                                unit_v.at[pl.ds(k * Ec, Ec)])

        def blk1(b, carry):
            base = b * _L
            vs = idxs[pl.ds(base, _L)]
            vd = idxd[pl.ds(base, _L)]
            gk = [g_v[pl.ds(k * Ec + base, _L)] for k in range(K)]

            def fl(f, acc):
                gs = plsc.load_gather(hs_v, [f * N + vs])
                gd = plsc.load_gather(hd_v, [f * N + vd])
                cc = gk[0] * wg_v[pl.ds(f * _L, _L)]
                for k in range(1, K):
                    cc += gk[k] * wg_v[pl.ds((k * FH + f) * _L, _L)]
                m = jnp.maximum(gs + gd + cc, 0.0)
                return acc + m * wa_v[pl.ds(f * _L, _L)]

            acc = lax.fori_loop(0, FH, fl, jnp.zeros((_L,), jnp.float32))
            l_v[pl.ds(base, _L)] = acc
            return carry
        lax.fori_loop(0, nb, blk1, 0)

        pltpu.sync_copy(l_v, sh.at[s])
        plsc.subcore_barrier()
        pltpu.sync_copy(sh.at[s + 1 - 2 * fhalf], lp_v)

        if final:
            _zero_vmem(upd_v, 192 // _L)

            @pl.when(fhalf == 0)
            def _():
                def blk2(b, carry):
                    base = b * _L
                    sc = l_v[pl.ds(base, _L)] + lp_v[pl.ds(base, _L)]
                    vd = idxd[pl.ds(base, _L)]
                    for k in range(3):
                        plsc.addupdate_scatter(
                            upd_v, [k * 64 + vd],
                            sc * unit_v[pl.ds(k * Ec + base, _L)])
                    return carry
                lax.fori_loop(0, nb, blk2, 0)
                pltpu.sync_copy(upd_v, out.at[chunk])
            return

        _zero_vmem(den_v, N // _L)

        def blkw(b, carry):
            base = b * _L
            w = jnp.exp(l_v[pl.ds(base, _L)] + lp_v[pl.ds(base, _L)])
            w_v[pl.ds(base, _L)] = w
            return carry
        lax.fori_loop(0, nb, blkw, 0)

        @pl.when(fhalf == 0)
        def _():
            def blkd(b, carry):
                base = b * _L
                plsc.addupdate_scatter(den_v, [idxd[pl.ds(base, _L)]],
                                       w_v[pl.ds(base, _L)])
                return carry
            lax.fori_loop(0, nb, blkd, 0)
            pltpu.sync_copy(den_v, den_out.at[chunk])

        _zero_vmem(u_v, FH * N // _L)

        def blk2(b, carry):
            base = b * _L
            vs = idxs[pl.ds(base, _L)]
            vd = idxd[pl.ds(base, _L)]
            wv = w_v[pl.ds(base, _L)]
            gk = [g_v[pl.ds(k * Ec + base, _L)] for k in range(K)]

            def fl(f, carry2):
                gs = plsc.load_gather(hs_v, [f * N + vs])
                gd = plsc.load_gather(hd_v, [f * N + vd])
                cc = gk[0] * wg_v[pl.ds(f * _L, _L)]
                for k in range(1, K):
                    cc += gk[k] * wg_v[pl.ds((k * FH + f) * _L, _L)]
                m = jnp.maximum(gs + gd + cc, 0.0)
                plsc.addupdate_scatter(u_v, [f * N + vd], m * wv)
                return carry2
            lax.fori_loop(0, FH, fl, 0)
            return carry
        lax.fori_loop(0, nb, blk2, 0)

        wid = c * 16 + s
        pltpu.sync_copy(u_v, u_out.at[wid])

    return pl.kernel(
        body, out_type=out_type, mesh=mesh, scratch_types=scratch,
        compiler_params=pltpu.CompilerParams(needs_layout_passes=False))


def _edge_aggregate(Hs, Hd, src, dst, G, Wg, wa, N):
    K, E = G.shape
    FO = Hs.shape[1]
    FH = FO // 2
    hsT = Hs.T.reshape(-1)
    hdT = Hd.T.reshape(-1)
    wg_rep = jnp.transpose(jnp.broadcast_to(
        Wg.reshape(K, 2, FH)[:, :, :, None], (K, 2, FH, _L)),
        (1, 0, 2, 3)).reshape(-1)
    wa_rep = jnp.broadcast_to(wa.reshape(FO, 1), (FO, _L)).reshape(-1)
    U_part, den_part = _sc_edge_pass(E, N, K, FO, False)(
        hsT, hdT, src, dst, G.reshape(-1), wg_rep, wa_rep)
    U = U_part.reshape(2, 8, 2, FH, N).sum(axis=(0, 1)).reshape(FO, N)
    den = den_part.sum(axis=0)
    return U.T, den


def _final_scatter(Hs, Hd, src, dst_local, G, Wg, wa, unitT, N):
    K, E = G.shape
    FO = Hs.shape[1]
    FH = FO // 2
    hsT = Hs.T.reshape(-1)
    hdT = Hd.T.reshape(-1)
    wg_rep = jnp.transpose(jnp.broadcast_to(
        Wg.reshape(K, 2, FH)[:, :, :, None], (K, 2, FH, _L)),
        (1, 0, 2, 3)).reshape(-1)
    wa_rep = jnp.broadcast_to(wa.reshape(FO, 1), (FO, _L)).reshape(-1)
    upd_part = _sc_edge_pass(E, N, K, FO, True)(
        hsT, hdT, src, dst_local, G.reshape(-1), wg_rep, wa_rep,
        unitT.reshape(-1))
    return upd_part.sum(axis=0).reshape(3, 64).T


def _mp_layer_sparse(h, src, dst, r, ewT, Wm, Wa, Wh, N, F):
    Hs = h @ Wm[0:F]
    Hd = h @ Wm[F:2 * F]
    G = jnp.concatenate([ewT, r[None, :]], axis=0)
    Wg = jnp.concatenate([Wm[2 * F:2 * F + 4], Wm[2 * F + 4][None]], axis=0)
    U, den = _edge_aggregate(Hs, Hd, src, dst, G, Wg, Wa[:, 0], N)
    agg = U / (den + (den == 0.0))[:, None]
    return jax.nn.relu(jnp.concatenate([h, agg], axis=-1) @ Wh)


def _mp_layer_combined(h, src_sp, dst_sp, G_sp, rr, Wm, Wa, Wh):
    N, F = REC_N + LIG_N, 128
    Hs = h @ Wm[0:F]
    Hd = h @ Wm[F:2 * F]
    Wet = Wm[2 * F:2 * F + 3]
    wr = Wm[2 * F + 3]
    Wg = jnp.stack([Wet[0], Wet[1], wr], axis=0)
    U, den = _edge_aggregate(Hs, Hd, src_sp, dst_sp, G_sp, Wg, Wa[:, 0], N)
    base = Wet[2] + rr[:, :, None] * wr
    m1 = jax.nn.relu(Hs[:REC_N, None, :] + Hd[None, REC_N:, :] + base)
    w1 = jnp.exp(m1 @ Wa[:, 0])
    m2 = jax.nn.relu(Hs[None, REC_N:, :] + Hd[:REC_N, None, :] + base)
    w2 = jnp.exp(m2 @ Wa[:, 0])
    den = den.at[REC_N:].add(w1.sum(axis=0)).at[:REC_N].add(w2.sum(axis=1))
    U = U.at[REC_N:].add(jnp.einsum('ij,ijf->jf', w1, m1))
    U = U.at[:REC_N].add(jnp.einsum('ij,ijf->if', w2, m2))
    agg = U / (den + (den == 0.0))[:, None]
    return jax.nn.relu(jnp.concatenate([h, agg], axis=-1) @ Wh)


def kernel(rec_x, rec_f, rec_vec, rec_edge_index, rec_edge_w, lig_x, lig_f,
           lig_edge_index, lig_edge_w, rWm0, rWa0, rWh0, rWm1, rWa1, rWh1,
           rWo, lWm0, lWa0, lWh0, lWm1, lWa1, lWh1, lWo, cWm0, cWa0, cWh0,
           cWm1, cWa1, cWh1, cWmf, cWdf):
    rs, rd = rec_edge_index[0], rec_edge_index[1]
    ls, ld = lig_edge_index[0], lig_edge_index[1]
    dr = rec_x[rd] - rec_x[rs]
    r_rec = jnp.sqrt(jnp.sum(dr * dr, -1) + 1e-12)
    dl = lig_x[ld] - lig_x[ls]
    r_lig = jnp.sqrt(jnp.sum(dl * dl, -1) + 1e-12)
    dd = lig_x[None, :, :] - rec_x[:, None, :]
    rr = jnp.sqrt(jnp.sum(dd * dd, -1) + 1e-12)

    vnorm = jnp.sqrt(jnp.sum(rec_vec[:, 0, :] ** 2, -1, keepdims=True)
                     + 1e-12)
    h = jnp.concatenate([rec_f, vnorm], axis=-1)
    rewT = rec_edge_w.T
    h = _mp_layer_sparse(h, rs, rd, r_rec, rewT, rWm0, rWa0, rWh0, REC_N, 129)
    h = _mp_layer_sparse(h, rs, rd, r_rec, rewT, rWm1, rWa1, rWh1, REC_N, 128)
    h_rec = h @ rWo
    h = lig_f
    lewT = lig_edge_w.T
    h = _mp_layer_sparse(h, ls, ld, r_lig, lewT, lWm0, lWa0, lWh0, LIG_N, 64)
    h = _mp_layer_sparse(h, ls, ld, r_lig, lewT, lWm1, lWa1, lWh1, LIG_N, 64)
    h_lig = h @ lWo

    hc = jnp.concatenate([h_rec, h_lig], axis=0)
    src_sp = jnp.concatenate([rs, ls + REC_N])
    dst_sp = jnp.concatenate([rd, ld + REC_N])
    Er, El = rs.shape[0], ls.shape[0]
    G_sp = jnp.stack([
        jnp.concatenate([jnp.ones(Er, jnp.float32), jnp.zeros(El, jnp.float32)]),
        jnp.concatenate([jnp.zeros(Er, jnp.float32), jnp.ones(El, jnp.float32)]),
        jnp.concatenate([r_rec, r_lig])], axis=0)
    hc = _mp_layer_combined(hc, src_sp, dst_sp, G_sp, rr, cWm0, cWa0, cWh0)
    hc = _mp_layer_combined(hc, src_sp, dst_sp, G_sp, rr, cWm1, cWa1, cWh1)

    F = 128
    Hs = hc @ cWmf[0:F]
    Hd = hc @ cWmf[F:2 * F]
    Wet = cWmf[2 * F:2 * F + 3]
    wr = cWmf[2 * F + 3]
    G_l = jnp.stack([jnp.ones(El, jnp.float32), r_lig], axis=0)
    Wg_l = jnp.stack([Wet[1], wr], axis=0)
    unit = dl / (r_lig[:, None] + 1e-9)
    upd_sp = _final_scatter(Hs, Hd, ls + REC_N, ld, G_l, Wg_l, cWdf[:, 0],
                            unit.T, REC_N + LIG_N)
    base = Wet[2] + rr[:, :, None] * wr
    mf1 = jax.nn.relu(Hs[:REC_N, None, :] + Hd[None, REC_N:, :] + base)
    sc1 = mf1 @ cWdf[:, 0]
    unit1 = dd / (rr[:, :, None] + 1e-9)
    upd = upd_sp + jnp.einsum('ij,ijc->jc', sc1, unit1)
    lig_new = lig_x + upd
    return jnp.stack([lig_x, lig_new])[None, :]
